# baseline (speedup 1.0000x reference)
import functools

import jax
import jax.numpy as jnp
from jax import lax
from jax.experimental import pallas as pl
from jax.experimental.pallas import tpu as pltpu
from jax.experimental.pallas import tpu_sc as plsc

_N = 10000
_D = 128
_E = 320000
_NC = 2
_NS = 16
_N_PAD = 10240
_ROWS_PER_TILE = _N_PAD // _NS
_EDGES_PER_TILE = _E // (_NC * _NS)
_CH = 80
_NCHUNK = _EDGES_PER_TILE // _CH
_NBUF = 5
_NGROUP = _NCHUNK // _NBUF


def _segment_sum_sc(hb, src, dst, zrows):
    mesh = plsc.VectorSubcoreMesh(core_axis_name="c", subcore_axis_name="s")

    @functools.partial(
        pl.kernel,
        out_type=jax.ShapeDtypeStruct((_NC, _N_PAD, _D), jnp.bfloat16),
        mesh=mesh,
        scratch_types=(
            [pltpu.VMEM_SHARED((_N_PAD, _D), jnp.bfloat16)]
            + [pltpu.VMEM((_CH,), jnp.int32) for _ in range(2 * _NBUF)]
            + [pltpu.VMEM((_CH, _D), jnp.bfloat16) for _ in range(_NBUF)]
            + [pltpu.SemaphoreType.DMA for _ in range(3 * _NBUF)]
        ),
        compiler_params=pltpu.CompilerParams(use_tc_tiling_on_sc=False),
    )
    def seg_kernel(h_hbm, src_hbm, dst_hbm, z_hbm, out_hbm, acc, *scratch):
        sidx = scratch[0:_NBUF]
        didx = scratch[_NBUF:2 * _NBUF]
        rows = scratch[2 * _NBUF:3 * _NBUF]
        semi = scratch[3 * _NBUF:4 * _NBUF]
        semg = scratch[4 * _NBUF:5 * _NBUF]
        sema = scratch[5 * _NBUF:6 * _NBUF]
        c = lax.axis_index("c")
        s = lax.axis_index("s")
        wid = c * _NS + s
        pltpu.sync_copy(z_hbm, acc.at[pl.ds(s * _ROWS_PER_TILE, _ROWS_PER_TILE)])
        plsc.subcore_barrier()
        ebase = wid * _EDGES_PER_TILE

        @pl.loop(0, _NGROUP)
        def _group(g):
            cbase = ebase + g * (_NBUF * _CH)
            idx_cp = []
            for b in range(_NBUF):
                @pl.when(g > 0)
                def _(b=b):
                    pltpu.make_async_copy(rows[b], acc.at[didx[b]], sema[b]).wait()
                off = cbase + b * _CH
                idx_cp.append(
                    (pltpu.async_copy(src_hbm.at[pl.ds(off, _CH)], sidx[b], semi[b]),
                     pltpu.async_copy(dst_hbm.at[pl.ds(off, _CH)], didx[b], semi[b])))
            g_cp = []
            for b in range(_NBUF):
                idx_cp[b][0].wait()
                idx_cp[b][1].wait()
                g_cp.append(pltpu.async_copy(h_hbm.at[sidx[b]], rows[b], semg[b]))
            for b in range(_NBUF):
                g_cp[b].wait()
                pltpu.async_copy(rows[b], acc.at[didx[b]], sema[b], add=True)

        for b in range(_NBUF):
            pltpu.make_async_copy(rows[b], acc.at[didx[b]], sema[b]).wait()
        plsc.subcore_barrier()
        pltpu.sync_copy(
            acc.at[pl.ds(s * _ROWS_PER_TILE, _ROWS_PER_TILE)],
            out_hbm.at[c, pl.ds(s * _ROWS_PER_TILE, _ROWS_PER_TILE)],
        )

    return seg_kernel(hb, src, dst, zrows)


_BLK = 1024


def _tc_layer(parts, hb_prev, w_rel, b_rel, w_root):
    def body(p_ref, h_ref, wr_ref, br_ref, wo_ref, o_ref):
        agg = p_ref[0].astype(jnp.float32) + p_ref[1].astype(jnp.float32)
        y = lax.dot_general(agg, wr_ref[...], (((1,), (1,)), ((), ())),
                            preferred_element_type=jnp.float32)
        h = h_ref[...].astype(jnp.float32)
        y = y + lax.dot_general(h, wo_ref[...], (((1,), (1,)), ((), ())),
                                preferred_element_type=jnp.float32)
        y = jnp.maximum(y + br_ref[...], 0.0)
        o_ref[...] = y.astype(jnp.bfloat16)

    return pl.pallas_call(
        body,
        grid=(_N_PAD // _BLK,),
        in_specs=[
            pl.BlockSpec((_NC, _BLK, _D), lambda i: (0, i, 0)),
            pl.BlockSpec((_BLK, _D), lambda i: (i, 0)),
            pl.BlockSpec((_D, _D), lambda i: (0, 0)),
            pl.BlockSpec((1, _D), lambda i: (0, 0)),
            pl.BlockSpec((_D, _D), lambda i: (0, 0)),
        ],
        out_specs=pl.BlockSpec((_BLK, _D), lambda i: (i, 0)),
        out_shape=jax.ShapeDtypeStruct((_N_PAD, _D), jnp.bfloat16),
    )(parts, hb_prev, w_rel, b_rel.reshape(1, _D), w_root)


_FBLK = 1000


def _tc_final(parts, hb_prev, w_rel, b_rel, w_root, w_lin, b_lin):
    def body(p_ref, h_ref, wr_ref, br_ref, wo_ref, wl_ref, bl_ref, o_ref):
        agg = p_ref[0].astype(jnp.float32) + p_ref[1].astype(jnp.float32)
        y = lax.dot_general(agg, wr_ref[...], (((1,), (1,)), ((), ())),
                            preferred_element_type=jnp.float32)
        h = h_ref[...].astype(jnp.float32)
        y = y + lax.dot_general(h, wo_ref[...], (((1,), (1,)), ((), ())),
                                preferred_element_type=jnp.float32)
        y = y + br_ref[...]
        z = lax.dot_general(y, wl_ref[...], (((1,), (1,)), ((), ())),
                            preferred_element_type=jnp.float32)
        o_ref[...] = z + bl_ref[...]

    return pl.pallas_call(
        body,
        grid=(_N // _FBLK,),
        in_specs=[
            pl.BlockSpec((_NC, _FBLK, _D), lambda i: (0, i, 0)),
            pl.BlockSpec((_FBLK, _D), lambda i: (i, 0)),
            pl.BlockSpec((_D, _D), lambda i: (0, 0)),
            pl.BlockSpec((1, _D), lambda i: (0, 0)),
            pl.BlockSpec((_D, _D), lambda i: (0, 0)),
            pl.BlockSpec((_D, _D), lambda i: (0, 0)),
            pl.BlockSpec((1, _D), lambda i: (0, 0)),
        ],
        out_specs=pl.BlockSpec((_FBLK, _D), lambda i: (i, 0)),
        out_shape=jax.ShapeDtypeStruct((_N, _D), jnp.float32),
    )(parts, hb_prev, w_rel, b_rel.reshape(1, _D), w_root, w_lin, b_lin.reshape(1, _D))


def kernel(x, edge_index, W_rel1, b_rel1, W_root1, W_rel2, b_rel2, W_root2,
           W_rel3, b_rel3, W_root3, W_lin, b_lin):
    src = edge_index[0]
    dst = edge_index[1]
    zrows = jnp.zeros((_ROWS_PER_TILE, _D), jnp.bfloat16)
    xb = jnp.pad(x, ((0, _N_PAD - _N), (0, 0))).astype(jnp.bfloat16)

    p1 = _segment_sum_sc(xb, src, dst, zrows)
    h1b = _tc_layer(p1, xb, W_rel1, b_rel1, W_root1)
    p2 = _segment_sum_sc(h1b, src, dst, zrows)
    h2b = _tc_layer(p2, h1b, W_rel2, b_rel2, W_root2)
    p3 = _segment_sum_sc(h2b, src, dst, zrows)
    return _tc_final(p3, h2b, W_rel3, b_rel3, W_root3, W_lin, b_lin)
